# baseline (speedup 1.0000x reference)
import functools

import jax
import jax.numpy as jnp
from jax import lax
from jax.experimental import pallas as pl
from jax.experimental.pallas import tpu as pltpu
from jax.experimental.pallas import tpu_sc as plsc

B = 8
T = 512
D = 384
F = 1600
TBL = B * T
ZERO_ROW = TBL

NC, NS = 2, 16
NW = NC * NS
CHUNK = 80
ROWS_FAST = 480
ROWS_SLOW = 320
FAST_CORE = 1
FAST_TOTAL = NS * ROWS_FAST


def _idx_body(pd_ref, batch_ref, idx_ref, table_ref):
    b = pl.program_id(0)

    @pl.when(b < B)
    def _():
        d = jnp.round(jnp.clip(pd_ref[...], 1.0, 20.0)).reshape(T, 1)
        rows = lax.broadcasted_iota(jnp.int32, (T, T), 0)
        cols = lax.broadcasted_iota(jnp.int32, (T, T), 1)
        tril = (rows >= cols).astype(jnp.float32)
        cum = jnp.dot(tril, d, preferred_element_type=jnp.float32)
        pos = lax.broadcasted_iota(jnp.int32, (T, F), 1).astype(jnp.float32)
        m = (pos >= cum).astype(jnp.float32)
        tok = jnp.dot(jnp.ones((1, T), jnp.float32), m,
                      preferred_element_type=jnp.float32)
        raw = tok.astype(jnp.int32)
        flat = jnp.where(raw < T, b * T + raw, ZERO_ROW)
        idx_ref[...] = flat.reshape(1, 1, F)
        table_ref[...] = batch_ref[...]

    @pl.when(b == B)
    def _():
        table_ref[...] = jnp.zeros_like(table_ref)


def _clampb(b):
    return jnp.minimum(b, B - 1)


_idx_call = pl.pallas_call(
    _idx_body,
    grid=(B + 1,),
    in_specs=[
        pl.BlockSpec((1, T, 1), lambda b: (_clampb(b), 0, 0)),
        pl.BlockSpec((1, T, D), lambda b: (_clampb(b), 0, 0)),
    ],
    out_specs=[
        pl.BlockSpec((1, 1, F), lambda b: (_clampb(b), 0, 0)),
        pl.BlockSpec((1, T, D), lambda b: (b, 0, 0)),
    ],
    out_shape=[
        jax.ShapeDtypeStruct((B, 1, F), jnp.int32),
        jax.ShapeDtypeStruct((B + 1, T, D), jnp.float32),
    ],
)


_sc_mesh = plsc.VectorSubcoreMesh(core_axis_name="c", subcore_axis_name="s")


@functools.partial(
    pl.kernel,
    mesh=_sc_mesh,
    out_type=jax.ShapeDtypeStruct((B * F, D), jnp.float32),
    scratch_types=[
        pltpu.VMEM((ROWS_FAST,), jnp.int32),
        pltpu.VMEM((CHUNK, D), jnp.float32),
        pltpu.VMEM((CHUNK, D), jnp.float32),
        pltpu.SemaphoreType.DMA,
        pltpu.SemaphoreType.DMA,
        pltpu.SemaphoreType.DMA,
        pltpu.SemaphoreType.DMA,
    ],
)
def _sc_gather(table_hbm, idx_hbm, out_hbm, idx_v, buf0, buf1,
               gsem0, gsem1, ssem0, ssem1):
    cid = lax.axis_index("c")
    sid = lax.axis_index("s")
    bufs = (buf0, buf1)
    gsems = (gsem0, gsem1)
    ssems = (ssem0, ssem1)

    def run(base, nrows):
        nch = nrows // CHUNK
        pltpu.sync_copy(idx_hbm.at[pl.ds(base, nrows)],
                        idx_v.at[pl.ds(0, nrows)])
        gcp = [None, None]
        scp = [None, None]
        gcp[0] = pltpu.async_copy(
            table_hbm.at[idx_v.at[pl.ds(0, CHUNK)]], buf0, gsem0)
        for c in range(nch):
            nxt = c + 1
            if nxt < nch:
                if c >= 1:
                    scp[nxt % 2].wait()
                gcp[nxt % 2] = pltpu.async_copy(
                    table_hbm.at[idx_v.at[pl.ds(nxt * CHUNK, CHUNK)]],
                    bufs[nxt % 2], gsems[nxt % 2])
            gcp[c % 2].wait()
            scp[c % 2] = pltpu.async_copy(
                bufs[c % 2], out_hbm.at[pl.ds(base + c * CHUNK, CHUNK)],
                ssems[c % 2])
        scp[(nch - 2) % 2].wait()
        scp[(nch - 1) % 2].wait()

    @pl.when(cid == FAST_CORE)
    def _():
        run(sid * ROWS_FAST, ROWS_FAST)

    @pl.when(cid != FAST_CORE)
    def _():
        run(FAST_TOTAL + sid * ROWS_SLOW, ROWS_SLOW)


def kernel(batch, predicted_durations):
    idx, table = _idx_call(predicted_durations, batch)
    out = _sc_gather(table.reshape((B + 1) * T, D), idx.reshape(B * F))
    return out.reshape(B, F, D)


# speedup vs baseline: 1.0939x; 1.0939x over previous
import functools

import jax
import jax.numpy as jnp
from jax import lax
from jax.experimental import pallas as pl
from jax.experimental.pallas import tpu as pltpu
from jax.experimental.pallas import tpu_sc as plsc

B = 8
T = 512
D = 384
F = 1600
TBL = B * T

NC, NS = 2, 16
NW = NC * NS
ROWS_PER_W = (B * F) // NW
CHUNK = 80
NCH = ROWS_PER_W // CHUNK
LPC = CHUNK // 16


def _idx_body(pd_ref, idx_ref):
    b = pl.program_id(0)
    d = jnp.round(jnp.clip(pd_ref[...], 1.0, 20.0)).reshape(T, 1)
    rows = lax.broadcasted_iota(jnp.int32, (T, T), 0)
    cols = lax.broadcasted_iota(jnp.int32, (T, T), 1)
    tril = (rows >= cols).astype(jnp.float32)
    cum = jnp.dot(tril, d, preferred_element_type=jnp.float32)
    pos = lax.broadcasted_iota(jnp.int32, (T, F), 1).astype(jnp.float32)
    m = (pos >= cum).astype(jnp.float32)
    tok = jnp.dot(jnp.ones((1, T), jnp.float32), m,
                  preferred_element_type=jnp.float32)
    raw = tok.astype(jnp.int32)
    flat = jnp.where(raw < T, b * T + raw, TBL)
    idx_ref[...] = flat.reshape(1, 1, F)


_idx_call = pl.pallas_call(
    _idx_body,
    grid=(B,),
    in_specs=[pl.BlockSpec((1, T, 1), lambda b: (b, 0, 0))],
    out_specs=pl.BlockSpec((1, 1, F), lambda b: (b, 0, 0)),
    out_shape=jax.ShapeDtypeStruct((B, 1, F), jnp.int32),
)


_sc_mesh = plsc.VectorSubcoreMesh(core_axis_name="c", subcore_axis_name="s")


@functools.partial(
    pl.kernel,
    mesh=_sc_mesh,
    out_type=jax.ShapeDtypeStruct((B * F, D), jnp.float32),
    scratch_types=[
        pltpu.VMEM((ROWS_PER_W,), jnp.int32),
        pltpu.VMEM((ROWS_PER_W,), jnp.int32),
        pltpu.VMEM((CHUNK, D), jnp.float32),
        pltpu.VMEM((CHUNK, D), jnp.float32),
        pltpu.SemaphoreType.DMA,
        pltpu.SemaphoreType.DMA,
        pltpu.SemaphoreType.DMA,
        pltpu.SemaphoreType.DMA,
    ],
)
def _sc_gather(table_hbm, idx_hbm, out_hbm, idx_v, idx2_v, buf0, buf1,
               gsem0, gsem1, ssem0, ssem1):
    wid = lax.axis_index("s") * NC + lax.axis_index("c")
    base = wid * ROWS_PER_W
    pltpu.sync_copy(idx_hbm.at[pl.ds(base, ROWS_PER_W)], idx_v)

    lanes = lax.iota(jnp.int32, 16)
    acc = jnp.zeros((16,), jnp.int32)
    for c in range(NCH):
        for k in range(LPC):
            off = c * CHUNK + k * 16
            v = idx_v[pl.ds(off, 16)]
            acc = acc + jnp.where(v < TBL, 1, 0).astype(jnp.int32)
            idx2_v[pl.ds(off, 16)] = jnp.minimum(v, TBL - 1)
    for sh in (8, 4, 2, 1):
        acc = acc + acc.at[(lanes + sh) % 16].get(mode="promise_in_bounds")
    nvalid = acc[0]
    ninv = [jnp.clip(nvalid - c * CHUNK, 0, CHUNK) for c in range(NCH)]

    bufs = (buf0, buf1)
    gsems = (gsem0, gsem1)
    ssems = (ssem0, ssem1)
    gcp = [None, None]
    scp = [None, None]

    def zero_tail(buf, zstart):
        @pl.when(zstart < CHUNK)
        def _():
            def zrow(r, carry):
                for k in range(D // 16):
                    buf[r, pl.ds(k * 16, 16)] = jnp.zeros((16,), jnp.float32)
                return carry
            lax.fori_loop(zstart, CHUNK, zrow, 0)

    gcp[0] = pltpu.async_copy(
        table_hbm.at[idx2_v.at[pl.ds(0, CHUNK)]], buf0, gsem0)
    for c in range(NCH):
        nxt = c + 1
        if nxt < NCH:
            if c >= 1:
                scp[nxt % 2].wait()
            gcp[nxt % 2] = pltpu.async_copy(
                table_hbm.at[idx2_v.at[pl.ds(nxt * CHUNK, CHUNK)]],
                bufs[nxt % 2], gsems[nxt % 2])
        gcp[c % 2].wait()
        zero_tail(bufs[c % 2], ninv[c])
        scp[c % 2] = pltpu.async_copy(
            bufs[c % 2], out_hbm.at[pl.ds(base + c * CHUNK, CHUNK)],
            ssems[c % 2])
    scp[(NCH - 2) % 2].wait()
    scp[(NCH - 1) % 2].wait()


def kernel(batch, predicted_durations):
    idx = _idx_call(predicted_durations)
    out = _sc_gather(batch.reshape(TBL, D), idx.reshape(B * F))
    return out.reshape(B, F, D)


# speedup vs baseline: 1.0943x; 1.0003x over previous
import functools

import jax
import jax.numpy as jnp
from jax import lax
from jax.experimental import pallas as pl
from jax.experimental.pallas import tpu as pltpu
from jax.experimental.pallas import tpu_sc as plsc

B = 8
T = 512
D = 384
F = 1600
TBL = B * T

NC, NS = 2, 16
NW = NC * NS
ROWS_PER_W = (B * F) // NW
CHUNK = 80
NCH = ROWS_PER_W // CHUNK
LPC = CHUNK // 16


def _idx_body(pd_ref, idx_ref):
    b = pl.program_id(0)
    d = jnp.round(jnp.clip(pd_ref[...], 1.0, 20.0)).reshape(T, 1)
    rows = lax.broadcasted_iota(jnp.int32, (T, T), 0)
    cols = lax.broadcasted_iota(jnp.int32, (T, T), 1)
    tril = (rows >= cols).astype(jnp.float32)
    cum = jnp.dot(tril, d, preferred_element_type=jnp.float32)
    pos = lax.broadcasted_iota(jnp.int32, (T, F), 1).astype(jnp.float32)
    m = (pos >= cum).astype(jnp.float32)
    tok = jnp.dot(jnp.ones((1, T), jnp.float32), m,
                  preferred_element_type=jnp.float32)
    raw = tok.astype(jnp.int32)
    flat = jnp.where(raw < T, b * T + raw, TBL)
    idx_ref[...] = flat.reshape(1, 1, F)


_idx_call = pl.pallas_call(
    _idx_body,
    grid=(B,),
    in_specs=[pl.BlockSpec((1, T, 1), lambda b: (b, 0, 0))],
    out_specs=pl.BlockSpec((1, 1, F), lambda b: (b, 0, 0)),
    out_shape=jax.ShapeDtypeStruct((B, 1, F), jnp.int32),
)


_sc_mesh = plsc.VectorSubcoreMesh(core_axis_name="c", subcore_axis_name="s")


@functools.partial(
    pl.kernel,
    mesh=_sc_mesh,
    out_type=jax.ShapeDtypeStruct((B * F, D), jnp.float32),
    scratch_types=[
        pltpu.VMEM((ROWS_PER_W,), jnp.int32),
        pltpu.VMEM((ROWS_PER_W,), jnp.int32),
        pltpu.VMEM((CHUNK, D), jnp.float32),
        pltpu.VMEM((CHUNK, D), jnp.float32),
        pltpu.SemaphoreType.DMA,
        pltpu.SemaphoreType.DMA,
        pltpu.SemaphoreType.DMA,
        pltpu.SemaphoreType.DMA,
    ],
)
def _sc_gather(table_hbm, idx_hbm, out_hbm, idx_v, idx2_v, buf0, buf1,
               gsem0, gsem1, ssem0, ssem1):
    wid = lax.axis_index("s") * NC + lax.axis_index("c")
    base = wid * ROWS_PER_W
    pltpu.sync_copy(idx_hbm.at[pl.ds(base, ROWS_PER_W)], idx_v)

    lanes = lax.iota(jnp.int32, 16)
    acc = jnp.zeros((16,), jnp.int32)
    for c in range(NCH):
        for k in range(LPC):
            off = c * CHUNK + k * 16
            v = idx_v[pl.ds(off, 16)]
            acc = acc + jnp.where(v < TBL, 1, 0).astype(jnp.int32)
            idx2_v[pl.ds(off, 16)] = jnp.minimum(v, TBL - 1)
    for sh in (8, 4, 2, 1):
        acc = acc + acc.at[(lanes + sh) % 16].get(mode="promise_in_bounds")
    nvalid = acc[0]
    zstarts = [jnp.clip(nvalid - c * CHUNK, 0, CHUNK) for c in range(NCH)]

    bufs = (buf0, buf1)
    gsems = (gsem0, gsem1)
    ssems = (ssem0, ssem1)
    gcp = [None, None]
    scp = [None, None]

    def zero_tail(buf, zstart):
        @pl.when(zstart < CHUNK)
        def _():
            def zrow(r, carry):
                for k in range(D // 16):
                    buf[r, pl.ds(k * 16, 16)] = jnp.zeros((16,), jnp.float32)
                return carry
            lax.fori_loop(zstart, CHUNK, zrow, 0)

    gcp[0] = pltpu.async_copy(
        table_hbm.at[idx2_v.at[pl.ds(0, CHUNK)]], buf0, gsem0)
    for c in range(NCH):
        nxt = c + 1
        if nxt < NCH:
            if c >= 1:
                scp[nxt % 2].wait()
            gcp[nxt % 2] = pltpu.async_copy(
                table_hbm.at[idx2_v.at[pl.ds(nxt * CHUNK, CHUNK)]],
                bufs[nxt % 2], gsems[nxt % 2])
        gcp[c % 2].wait()
        zero_tail(bufs[c % 2], zstarts[c])
        scp[c % 2] = pltpu.async_copy(
            bufs[c % 2], out_hbm.at[pl.ds(base + c * CHUNK, CHUNK)],
            ssems[c % 2])
    scp[(NCH - 2) % 2].wait()
    scp[(NCH - 1) % 2].wait()


def kernel(batch, predicted_durations):
    idx = _idx_call(predicted_durations)
    out = _sc_gather(batch.reshape(TBL, D), idx.reshape(B * F))
    return out.reshape(B, F, D)
